# baseline (speedup 1.0000x reference)
import functools

import jax
import jax.numpy as jnp
from jax import lax
from jax.experimental import pallas as pl
from jax.experimental.pallas import tpu as pltpu
from jax.experimental.pallas import tpu_sc as plsc

B, L, D = 1024, 50, 64
NLP = 768
NR = 4
BL = B * L

NC, NS = 2, 16
NW = NC * NS
CB = 80
CPW = BL // (NW * CB)
NG = 4
NBUF = 6
LAG = 2


def _sc_gather2(exe_t, skill_t, idx_all):
    mesh = plsc.VectorSubcoreMesh(core_axis_name="c", subcore_axis_name="s")

    @functools.partial(
        pl.kernel,
        mesh=mesh,
        out_type=[jax.ShapeDtypeStruct((BL, D), jnp.float32)] * 2,
        scratch_types=[
            pltpu.VMEM((NG, CPW, CB), jnp.int32),
            *[pltpu.VMEM((CB, D), jnp.float32) for _ in range(NBUF)],
            *[pltpu.SemaphoreType.DMA for _ in range(2 * NBUF)],
        ],
        compiler_params=pltpu.CompilerParams(use_tc_tiling_on_sc=False),
    )
    def k(exe_hbm, skill_hbm, idx_hbm, genc, gout, idx_v, *bufs_and_sems):
        bufs = bufs_and_sems[:NBUF]
        gsem = bufs_and_sems[NBUF:2 * NBUF]
        wsem = bufs_and_sems[2 * NBUF:]
        wid = lax.axis_index("s") * NC + lax.axis_index("c")
        base = wid * (CPW * CB)

        pltpu.sync_copy(idx_hbm.at[wid], idx_v)

        jobs = []
        for j in range(CPW):
            jobs.append((0, 1, genc, j))
            jobs.append((2, 3, gout, j))
        nj = len(jobs)
        h1 = [None] * nj
        h2 = [None] * nj
        hw = [None] * nj

        for i in range(nj + 2 * LAG):
            if i < nj:
                s = i % NBUF
                if i >= NBUF:
                    hw[i - NBUF].wait()
                ge, _, _, j = jobs[i]
                h1[i] = pltpu.async_copy(
                    exe_hbm.at[idx_v.at[ge, j]], bufs[s], gsem[s])
            if LAG <= i < nj + LAG:
                t = i - LAG
                s = t % NBUF
                _, gs, _, j = jobs[t]
                h1[t].wait()
                h2[t] = pltpu.async_copy(
                    skill_hbm.at[idx_v.at[gs, j]], bufs[s], gsem[s],
                    add=True)
            if i >= 2 * LAG:
                t = i - 2 * LAG
                s = t % NBUF
                _, _, dst, j = jobs[t]
                h2[t].wait()
                start = pl.multiple_of(base + j * CB, 8)
                hw[t] = pltpu.async_copy(
                    bufs[s], dst.at[pl.ds(start, CB)], wsem[s])
        for t in range(nj - NBUF, nj):
            hw[t].wait()

    return k(exe_t, skill_t, idx_all)


def _dense_body(x1_ref, x2_ref, el_ref, r_ref, pos_ref, wn_ref, bn_ref,
                wt_ref, bt_ref, resp_ref, encp_ref, dec_ref, outp_ref):
    w = wn_ref[...]
    pos_l = pos_ref[...].reshape(1, D)
    bn = bn_ref[...]

    x1 = x1_ref[...].reshape(B, NLP)
    y1 = jnp.dot(x1, w, preferred_element_type=jnp.float32)
    encp_ref[...] = (y1 + bn + pos_l).T.reshape(1, D, B)

    el = el_ref[...].reshape(B, 1)
    r = r_ref[...].reshape(B, 1)
    onehot = (r == lax.broadcasted_iota(jnp.int32, (1, NR), 1)
              ).astype(jnp.float32)
    dec_r = jnp.dot(onehot, resp_ref[...], preferred_element_type=jnp.float32)
    dec = el * wt_ref[...] + bt_ref[...] + pos_l + dec_r
    dec_ref[...] = dec.T.reshape(1, D, B)

    x2 = x2_ref[...].reshape(B, NLP)
    y2 = jnp.dot(x2, w, preferred_element_type=jnp.float32)
    outp_ref[...] = (y2 + bn).T.reshape(1, D, B)


def _tc_dense(x1_t, x2_t, el_t, r_t, pos, w_nlp, b_nlp, w_time, b_time,
              resp):
    big = pl.BlockSpec((1, B, NLP), lambda i: (i, 0, 0))
    row = pl.BlockSpec((1, 1, B), lambda i: (i, 0, 0))
    tokT = pl.BlockSpec((1, D, B), lambda i: (i, 0, 0))
    return pl.pallas_call(
        _dense_body,
        grid=(L,),
        in_specs=[
            big, big, row, row,
            pl.BlockSpec((1, 1, D), lambda i: (i, 0, 0)),
            pl.BlockSpec((NLP, D), lambda i: (0, 0)),
            pl.BlockSpec((1, D), lambda i: (0, 0)),
            pl.BlockSpec((1, D), lambda i: (0, 0)),
            pl.BlockSpec((1, D), lambda i: (0, 0)),
            pl.BlockSpec((NR, D), lambda i: (0, 0)),
        ],
        out_specs=[tokT, tokT, tokT],
        out_shape=[jax.ShapeDtypeStruct((L, D, B), jnp.float32)] * 3,
    )(x1_t, x2_t, el_t, r_t, pos, w_nlp, b_nlp, w_time, b_time, resp)


def _final_body(encp_ref, outp_ref, ge_ref, go_ref, enc_ref, out_ref):
    def unpack_t(vref):
        t = vref[...].reshape(512, 128).T
        return jnp.stack([t[:D, :], t[D:, :]], axis=2).reshape(D, B)

    enc_ref[...] = (encp_ref[...].reshape(D, B)
                    + unpack_t(ge_ref)).reshape(1, D, B)
    out_ref[...] = (outp_ref[...].reshape(D, B)
                    + unpack_t(go_ref)).reshape(1, D, B)


def _tc_final(encp, outp, g_enc, g_out):
    tokT = pl.BlockSpec((1, D, B), lambda i: (i, 0, 0))
    pair = pl.BlockSpec((1, B // 2, 2 * D), lambda i: (i, 0, 0))
    return pl.pallas_call(
        _final_body,
        grid=(L,),
        in_specs=[tokT, tokT, pair, pair],
        out_specs=[tokT, tokT],
        out_shape=[jax.ShapeDtypeStruct((L, D, B), jnp.float32)] * 2,
    )(encp, outp, g_enc, g_out)


def kernel(input_nlp_embedding, input_exercise, input_skill, input_r,
           in_elapsed_time, output_nlp_embedding, out_exercise, out_skill,
           exercise_table, skill_table, response_table, position_table,
           W_time, b_time, W_nlp, b_nlp):
    def idx_t(a):
        return a.astype(jnp.int32).T.reshape(NW, CPW, CB)

    idx_all = jnp.stack(
        [idx_t(input_exercise), idx_t(input_skill),
         idx_t(out_exercise), idx_t(out_skill)], axis=1)

    g_enc, g_out = _sc_gather2(exercise_table, skill_table, idx_all)

    encp, dec_t, outp = _tc_dense(
        input_nlp_embedding.transpose(1, 0, 2),
        output_nlp_embedding.transpose(1, 0, 2),
        in_elapsed_time[:, :, 0].T.reshape(L, 1, B),
        input_r.astype(jnp.int32).T.reshape(L, 1, B),
        position_table.reshape(L, 1, D), W_nlp, b_nlp.reshape(1, D),
        W_time, b_time.reshape(1, D), response_table)

    enc_t, out_t = _tc_final(
        encp, outp, g_enc.reshape(L, B // 2, 2 * D),
        g_out.reshape(L, B // 2, 2 * D))

    return (enc_t.transpose(2, 0, 1), dec_t.transpose(2, 0, 1),
            out_t.transpose(2, 0, 1))


# speedup vs baseline: 2.4797x; 2.4797x over previous
import functools

import jax
import jax.numpy as jnp
from jax import lax
from jax.experimental import pallas as pl
from jax.experimental.pallas import tpu as pltpu
from jax.experimental.pallas import tpu_sc as plsc

B, L, D = 1024, 50, 64
NLP = 768
NR = 4
BL = B * L

NC, NS = 2, 16
NW = NC * NS
CB = 80
CPW = BL // (NW * CB)
NG = 4
NBUF = 6
LAG = 2


def _sc_gather2(exe_t, skill_t, idx_all):
    mesh = plsc.VectorSubcoreMesh(core_axis_name="c", subcore_axis_name="s")

    @functools.partial(
        pl.kernel,
        mesh=mesh,
        out_type=[jax.ShapeDtypeStruct((BL, D), jnp.float32)] * 2,
        scratch_types=[
            pltpu.VMEM((NG, CPW, CB), jnp.int32),
            *[pltpu.VMEM((CB, D), jnp.float32) for _ in range(NBUF)],
            *[pltpu.SemaphoreType.DMA for _ in range(2 * NBUF)],
        ],
        compiler_params=pltpu.CompilerParams(use_tc_tiling_on_sc=False),
    )
    def k(exe_hbm, skill_hbm, idx_hbm, genc, gout, idx_v, *bufs_and_sems):
        bufs = bufs_and_sems[:NBUF]
        gsem = bufs_and_sems[NBUF:2 * NBUF]
        wsem = bufs_and_sems[2 * NBUF:]
        wid = lax.axis_index("s") * NC + lax.axis_index("c")
        base = wid * (CPW * CB)

        pltpu.sync_copy(idx_hbm.at[wid], idx_v)

        jobs = []
        for j in range(CPW):
            jobs.append((0, 1, genc, j))
            jobs.append((2, 3, gout, j))
        nj = len(jobs)
        h1 = [None] * nj
        h2 = [None] * nj
        hw = [None] * nj

        for i in range(nj + 2 * LAG):
            if i < nj:
                s = i % NBUF
                if i >= NBUF:
                    hw[i - NBUF].wait()
                ge, _, _, j = jobs[i]
                h1[i] = pltpu.async_copy(
                    exe_hbm.at[idx_v.at[ge, j]], bufs[s], gsem[s])
            if LAG <= i < nj + LAG:
                t = i - LAG
                s = t % NBUF
                _, gs, _, j = jobs[t]
                h1[t].wait()
                h2[t] = pltpu.async_copy(
                    skill_hbm.at[idx_v.at[gs, j]], bufs[s], gsem[s],
                    add=True)
            if i >= 2 * LAG:
                t = i - 2 * LAG
                s = t % NBUF
                _, _, dst, j = jobs[t]
                h2[t].wait()
                start = pl.multiple_of(base + j * CB, 8)
                hw[t] = pltpu.async_copy(
                    bufs[s], dst.at[pl.ds(start, CB)], wsem[s])
        for t in range(nj - NBUF, nj):
            hw[t].wait()

    return k(exe_t, skill_t, idx_all)


def _dense_body(x1_ref, x2_ref, el_ref, r_ref, pos_ref, wn_ref, bn_ref,
                wt_ref, bt_ref, resp_ref, encp_ref, dec_ref, outp_ref):
    w = wn_ref[...]
    pos_l = pos_ref[...].reshape(1, D)
    bn = bn_ref[...]

    x1 = x1_ref[...].reshape(B, NLP)
    y1 = jnp.dot(x1, w, preferred_element_type=jnp.float32)
    encp_ref[...] = (y1 + bn + pos_l).T.reshape(1, D, B)

    el = el_ref[...].reshape(B, 1)
    r = r_ref[...].reshape(B, 1)
    onehot = (r == lax.broadcasted_iota(jnp.int32, (1, NR), 1)
              ).astype(jnp.float32)
    dec_r = jnp.dot(onehot, resp_ref[...], preferred_element_type=jnp.float32)
    dec = el * wt_ref[...] + bt_ref[...] + pos_l + dec_r
    dec_ref[...] = dec.T.reshape(1, D, B)

    x2 = x2_ref[...].reshape(B, NLP)
    y2 = jnp.dot(x2, w, preferred_element_type=jnp.float32)
    outp_ref[...] = (y2 + bn).T.reshape(1, D, B)


def _tc_dense(x1_t, x2_t, el_t, r_t, pos, w_nlp, b_nlp, w_time, b_time,
              resp):
    big = pl.BlockSpec((1, B, NLP), lambda i: (i, 0, 0))
    row = pl.BlockSpec((1, 1, B), lambda i: (i, 0, 0))
    tokT = pl.BlockSpec((1, D, B), lambda i: (i, 0, 0))
    return pl.pallas_call(
        _dense_body,
        grid=(L,),
        in_specs=[
            big, big, row, row,
            pl.BlockSpec((1, 1, D), lambda i: (i, 0, 0)),
            pl.BlockSpec((NLP, D), lambda i: (0, 0)),
            pl.BlockSpec((1, D), lambda i: (0, 0)),
            pl.BlockSpec((1, D), lambda i: (0, 0)),
            pl.BlockSpec((1, D), lambda i: (0, 0)),
            pl.BlockSpec((NR, D), lambda i: (0, 0)),
        ],
        out_specs=[tokT, tokT, tokT],
        out_shape=[jax.ShapeDtypeStruct((L, D, B), jnp.float32)] * 3,
    )(x1_t, x2_t, el_t, r_t, pos, w_nlp, b_nlp, w_time, b_time, resp)


def _final_body(encp_ref, outp_ref, ge_ref, go_ref, enc_ref, out_ref):
    ge = ge_ref[...].reshape(B, D)
    enc_ref[...] = (encp_ref[...].reshape(D, B) + ge.T).reshape(1, D, B)
    go = go_ref[...].reshape(B, D)
    out_ref[...] = (outp_ref[...].reshape(D, B) + go.T).reshape(1, D, B)


def _tc_final(encp, outp, g_enc, g_out):
    tokT = pl.BlockSpec((1, D, B), lambda i: (i, 0, 0))
    tok = pl.BlockSpec((1, B, D), lambda i: (i, 0, 0))
    return pl.pallas_call(
        _final_body,
        grid=(L,),
        in_specs=[tokT, tokT, tok, tok],
        out_specs=[tokT, tokT],
        out_shape=[jax.ShapeDtypeStruct((L, D, B), jnp.float32)] * 2,
    )(encp, outp, g_enc, g_out)


def kernel(input_nlp_embedding, input_exercise, input_skill, input_r,
           in_elapsed_time, output_nlp_embedding, out_exercise, out_skill,
           exercise_table, skill_table, response_table, position_table,
           W_time, b_time, W_nlp, b_nlp):
    def idx_t(a):
        return a.astype(jnp.int32).T.reshape(NW, CPW, CB)

    idx_all = jnp.stack(
        [idx_t(input_exercise), idx_t(input_skill),
         idx_t(out_exercise), idx_t(out_skill)], axis=1)

    g_enc, g_out = _sc_gather2(exercise_table, skill_table, idx_all)

    encp, dec_t, outp = _tc_dense(
        input_nlp_embedding.transpose(1, 0, 2),
        output_nlp_embedding.transpose(1, 0, 2),
        in_elapsed_time[:, :, 0].T.reshape(L, 1, B),
        input_r.astype(jnp.int32).T.reshape(L, 1, B),
        position_table.reshape(L, 1, D), W_nlp, b_nlp.reshape(1, D),
        W_time, b_time.reshape(1, D), response_table)

    enc_t, out_t = _tc_final(
        encp, outp, g_enc.reshape(L, B, D), g_out.reshape(L, B, D))

    return (enc_t.transpose(2, 0, 1), dec_t.transpose(2, 0, 1),
            out_t.transpose(2, 0, 1))


# speedup vs baseline: 2.5841x; 1.0421x over previous
import functools

import jax
import jax.numpy as jnp
from jax import lax
from jax.experimental import pallas as pl
from jax.experimental.pallas import tpu as pltpu
from jax.experimental.pallas import tpu_sc as plsc

B, L, D = 1024, 50, 64
NLP = 768
NR = 4
BL = B * L

NC, NS = 2, 16
NW = NC * NS
CB = 80
CPW = BL // (NW * CB)
NG = 4
NBUF = 6
LAG = 2


def _sc_gather2(exe_t, skill_t, idx_all):
    mesh = plsc.VectorSubcoreMesh(core_axis_name="c", subcore_axis_name="s")

    @functools.partial(
        pl.kernel,
        mesh=mesh,
        out_type=[jax.ShapeDtypeStruct((BL, D), jnp.float32)] * 2,
        scratch_types=[
            pltpu.VMEM((NG, CPW, CB), jnp.int32),
            *[pltpu.VMEM((CB, D), jnp.float32) for _ in range(NBUF)],
            *[pltpu.SemaphoreType.DMA for _ in range(2 * NBUF)],
        ],
        compiler_params=pltpu.CompilerParams(use_tc_tiling_on_sc=False),
    )
    def k(exe_hbm, skill_hbm, idx_hbm, genc, gout, idx_v, *bufs_and_sems):
        bufs = bufs_and_sems[:NBUF]
        gsem = bufs_and_sems[NBUF:2 * NBUF]
        wsem = bufs_and_sems[2 * NBUF:]
        wid = lax.axis_index("s") * NC + lax.axis_index("c")
        base = wid * (CPW * CB)

        pltpu.sync_copy(idx_hbm.at[wid], idx_v)

        jobs = []
        for j in range(CPW):
            jobs.append((0, 1, genc, j))
            jobs.append((2, 3, gout, j))
        nj = len(jobs)
        h1 = [None] * nj
        h2 = [None] * nj
        hw = [None] * nj

        for i in range(nj + 2 * LAG):
            if i < nj:
                s = i % NBUF
                if i >= NBUF:
                    hw[i - NBUF].wait()
                ge, _, _, j = jobs[i]
                h1[i] = pltpu.async_copy(
                    exe_hbm.at[idx_v.at[ge, j]], bufs[s], gsem[s])
            if LAG <= i < nj + LAG:
                t = i - LAG
                s = t % NBUF
                _, gs, _, j = jobs[t]
                h1[t].wait()
                h2[t] = pltpu.async_copy(
                    skill_hbm.at[idx_v.at[gs, j]], bufs[s], gsem[s],
                    add=True)
            if i >= 2 * LAG:
                t = i - 2 * LAG
                s = t % NBUF
                _, _, dst, j = jobs[t]
                h2[t].wait()
                start = pl.multiple_of(base + j * CB, 8)
                hw[t] = pltpu.async_copy(
                    bufs[s], dst.at[pl.ds(start, CB)], wsem[s])
        for t in range(nj - NBUF, nj):
            hw[t].wait()

    return k(exe_t, skill_t, idx_all)


BLK = 2


def _tc_body(x1_ref, x2_ref, el_ref, r_ref, ge_ref, go_ref,
             pos_ref, wn_ref, bn_ref, wt_ref, bt_ref, resp_ref,
             enc_ref, dec_ref, out_ref):
    w = wn_ref[...]
    bn = bn_ref[...]
    for k in range(BLK):
        pos_l = pos_ref[k].reshape(1, D)

        x1 = x1_ref[k].reshape(B, NLP)
        y1 = jnp.dot(x1, w, preferred_element_type=jnp.float32)
        enc = y1 + bn + pos_l + ge_ref[k].reshape(B, D)
        enc_ref[k] = enc.T

        el = el_ref[k].reshape(B, 1)
        r = r_ref[k].reshape(B, 1)
        onehot = (r == lax.broadcasted_iota(jnp.int32, (1, NR), 1)
                  ).astype(jnp.float32)
        dec_r = jnp.dot(onehot, resp_ref[...],
                        preferred_element_type=jnp.float32)
        dec = el * wt_ref[...] + bt_ref[...] + pos_l + dec_r
        dec_ref[k] = dec.T

        x2 = x2_ref[k].reshape(B, NLP)
        y2 = jnp.dot(x2, w, preferred_element_type=jnp.float32)
        out = y2 + bn + go_ref[k].reshape(B, D)
        out_ref[k] = out.T


def _tc_combine(x1_t, x2_t, el_t, r_t, g_enc, g_out,
                pos, w_nlp, b_nlp, w_time, b_time, resp):
    big = pl.BlockSpec((BLK, B, NLP), lambda i: (i, 0, 0))
    tok = pl.BlockSpec((BLK, B, D), lambda i: (i, 0, 0))
    row = pl.BlockSpec((BLK, 1, B), lambda i: (i, 0, 0))
    tokT = pl.BlockSpec((BLK, D, B), lambda i: (i, 0, 0))
    return pl.pallas_call(
        _tc_body,
        grid=(L // BLK,),
        in_specs=[
            big, big, row, row,
            tok, tok,
            pl.BlockSpec((BLK, 1, D), lambda i: (i, 0, 0)),
            pl.BlockSpec((NLP, D), lambda i: (0, 0)),
            pl.BlockSpec((1, D), lambda i: (0, 0)),
            pl.BlockSpec((1, D), lambda i: (0, 0)),
            pl.BlockSpec((1, D), lambda i: (0, 0)),
            pl.BlockSpec((NR, D), lambda i: (0, 0)),
        ],
        out_specs=[tokT, tokT, tokT],
        out_shape=[jax.ShapeDtypeStruct((L, D, B), jnp.float32)] * 3,
        compiler_params=pltpu.CompilerParams(
            vmem_limit_bytes=100 * 1024 * 1024),
    )(x1_t, x2_t, el_t, r_t, g_enc, g_out,
      pos, w_nlp, b_nlp, w_time, b_time, resp)


def kernel(input_nlp_embedding, input_exercise, input_skill, input_r,
           in_elapsed_time, output_nlp_embedding, out_exercise, out_skill,
           exercise_table, skill_table, response_table, position_table,
           W_time, b_time, W_nlp, b_nlp):
    def idx_t(a):
        return a.astype(jnp.int32).T.reshape(NW, CPW, CB)

    idx_all = jnp.stack(
        [idx_t(input_exercise), idx_t(input_skill),
         idx_t(out_exercise), idx_t(out_skill)], axis=1)

    g_enc, g_out = _sc_gather2(exercise_table, skill_table, idx_all)

    def tok3d(a):
        return a.reshape(L, B, D)

    enc_t, dec_t, out_t = _tc_combine(
        input_nlp_embedding.transpose(1, 0, 2),
        output_nlp_embedding.transpose(1, 0, 2),
        in_elapsed_time[:, :, 0].T.reshape(L, 1, B),
        input_r.astype(jnp.int32).T.reshape(L, 1, B),
        tok3d(g_enc), tok3d(g_out),
        position_table.reshape(L, 1, D), W_nlp, b_nlp.reshape(1, D), W_time,
        b_time.reshape(1, D), response_table)
    return (enc_t.transpose(2, 0, 1), dec_t.transpose(2, 0, 1),
            out_t.transpose(2, 0, 1))
